# baseline (speedup 1.0000x reference)
import functools

import jax
import jax.numpy as jnp
from jax import lax
from jax.experimental import pallas as pl
from jax.experimental.pallas import tpu as pltpu
from jax.experimental.pallas import tpu_sc as plsc

NC = 2
NS = 16
NW = NC * NS
LANES = 16
SEG = 104
POFF = (0, 96)
NQ = 4


def _build(B, T, V, D):
    assert B % NW == 0
    RPW = B // NW
    assert RPW >= 8 and (RPW - 1 - 3) % NQ == 0
    assert SEG + POFF[1] == T and POFF[1] % 8 == 0

    mesh = plsc.VectorSubcoreMesh(
        core_axis_name="c", subcore_axis_name="s", num_cores=NC, num_subcores=NS
    )

    @functools.partial(
        pl.kernel,
        out_type=jax.ShapeDtypeStruct((B, T, D), jnp.float32),
        mesh=mesh,
        scratch_types=[
            pltpu.VMEM((RPW, 2, SEG), jnp.int32),
            pltpu.VMEM_SHARED((T, D), jnp.float32),
            pltpu.VMEM((2, NQ, SEG, D), jnp.float32),
            pltpu.SemaphoreType.DMA((2, NQ)),
            pltpu.SemaphoreType.DMA((2, NQ)),
            pltpu.SemaphoreType.DMA((2, NQ)),
        ],
    )
    def emb(x_hbm, tok_hbm, pos_hbm, out_hbm, idx_v, pos_v, buf,
            rsem, gsem, ssem):
        wid = lax.axis_index("s") * NC + lax.axis_index("c")
        pltpu.sync_copy(x_hbm.at[wid], idx_v)

        @pl.when(lax.axis_index("s") == 0)
        def _stage_pos():
            pltpu.sync_copy(pos_hbm, pos_v)

        plsc.subcore_barrier()
        rbase = wid * RPW

        def pos_src(b):
            return pos_v.at[pl.ds(POFF[b], SEG)]

        def issue_refill(b, q):
            pltpu.async_copy(pos_src(b), buf.at[b, q], rsem.at[b, q])

        def wait_refill(b, q):
            pltpu.make_async_copy(
                pos_src(b), buf.at[b, q], rsem.at[b, q]).wait()

        def issue_gather(r, b, q):
            pltpu.async_copy(
                tok_hbm.at[idx_v.at[r, b]], buf.at[b, q], gsem.at[b, q],
                add=True)

        def wait_gather(r, b, q):
            pltpu.make_async_copy(
                tok_hbm.at[idx_v.at[r, b]], buf.at[b, q], gsem.at[b, q]).wait()

        def issue_store(r, b, q):
            pltpu.async_copy(
                buf.at[b, q], out_hbm.at[rbase + r, pl.ds(POFF[b], SEG)],
                ssem.at[b, q])

        def wait_store(r, b, q):
            pltpu.make_async_copy(
                buf.at[b, q], out_hbm.at[rbase + r, pl.ds(POFF[b], SEG)],
                ssem.at[b, q]).wait()

        def prep(r, b, q):
            wait_refill(b, q)
            issue_gather(r, b, q)

        for b in range(2):
            for q in range(NQ):
                issue_refill(b, q)
        for b in range(2):
            prep(0, b, 0)
        for b in range(2):
            prep(1, b, 1)
        for b in range(2):
            wait_gather(0, b, 0)
            issue_store(0, b, 0)
            prep(2, b, 2)
        for b in range(2):
            wait_gather(1, b, 1)
            issue_store(1, b, 1)
        for b in range(2):
            prep(3, b, 3)

        def step(r, q0, q2):
            for b in range(2):
                wait_gather(r - 1, b, q0)
                issue_store(r - 1, b, q0)
            for b in range(2):
                wait_store(r - 3, b, q2)
                issue_refill(b, q2)
            for b in range(2):
                prep(r + 1, b, q2)

        @pl.loop(3, RPW - 1, step=NQ)
        def _body(r0):
            for dr in range(NQ):
                step(r0 + dr, (dr + 2) % NQ, dr % NQ)

        for b in range(2):
            wait_gather(RPW - 2, b, (RPW - 2) % NQ)
            issue_store(RPW - 2, b, (RPW - 2) % NQ)
        for b in range(2):
            wait_gather(RPW - 1, b, (RPW - 1) % NQ)
            issue_store(RPW - 1, b, (RPW - 1) % NQ)
        for r in range(RPW - NQ, RPW):
            for b in range(2):
                wait_store(r, b, r % NQ)

    return emb


def kernel(x, token_table, pos_table):
    B, T = x.shape
    V, D = token_table.shape
    emb = _build(B, T, V, D)
    xi = x.astype(jnp.int32)
    segs = jnp.stack([xi[:, 0:SEG], xi[:, POFF[1]:T]], axis=1)
    x_seg = segs.reshape(NW, B // NW, 2, SEG)
    return emb(x_seg, token_table, pos_table)


# speedup vs baseline: 1.0154x; 1.0154x over previous
import functools

import jax
import jax.numpy as jnp
from jax import lax
from jax.experimental import pallas as pl
from jax.experimental.pallas import tpu as pltpu
from jax.experimental.pallas import tpu_sc as plsc

NC = 2
NS = 16
NW = NC * NS
LANES = 16
SEG = 104
POFF = (0, 96)
NQ = 3


def _build(B, T, V, D):
    assert B % NW == 0
    RPW = B // NW
    assert RPW >= 8 and (RPW - 5) % NQ == 0
    assert SEG + POFF[1] == T and POFF[1] % 8 == 0

    mesh = plsc.VectorSubcoreMesh(
        core_axis_name="c", subcore_axis_name="s", num_cores=NC, num_subcores=NS
    )

    @functools.partial(
        pl.kernel,
        out_type=jax.ShapeDtypeStruct((B, T, D), jnp.float32),
        mesh=mesh,
        scratch_types=[
            pltpu.VMEM((RPW, 2, SEG), jnp.int32),
            pltpu.VMEM_SHARED((T, D), jnp.float32),
            pltpu.VMEM((2, NQ, SEG, D), jnp.float32),
            pltpu.SemaphoreType.DMA((2, NQ)),
            pltpu.SemaphoreType.DMA((2, NQ)),
            pltpu.SemaphoreType.DMA((2, NQ)),
        ],
    )
    def emb(x_hbm, tok_hbm, pos_hbm, out_hbm, idx_v, pos_v, buf,
            rsem, gsem, ssem):
        sid = lax.axis_index("s")
        wid = sid * NC + lax.axis_index("c")
        PCH = 40

        @pl.when(sid < T // PCH)
        def _stage_pos():
            sl = pl.ds(sid * PCH, PCH)
            pltpu.sync_copy(pos_hbm.at[sl], pos_v.at[sl])

        pltpu.sync_copy(x_hbm.at[wid], idx_v)
        plsc.subcore_barrier()
        rbase = wid * RPW

        def pos_src(b):
            return pos_v.at[pl.ds(POFF[b], SEG)]

        def issue_refill(b, q):
            pltpu.async_copy(pos_src(b), buf.at[b, q], rsem.at[b, q])

        def wait_refill(b, q):
            pltpu.make_async_copy(
                pos_src(b), buf.at[b, q], rsem.at[b, q]).wait()

        def issue_gather(r, b, q):
            pltpu.async_copy(
                tok_hbm.at[idx_v.at[r, b]], buf.at[b, q], gsem.at[b, q],
                add=True)

        def wait_gather(r, b, q):
            pltpu.make_async_copy(
                tok_hbm.at[idx_v.at[r, b]], buf.at[b, q], gsem.at[b, q]).wait()

        def issue_store(r, b, q):
            pltpu.async_copy(
                buf.at[b, q], out_hbm.at[rbase + r, pl.ds(POFF[b], SEG)],
                ssem.at[b, q])

        def wait_store(r, b, q):
            pltpu.make_async_copy(
                buf.at[b, q], out_hbm.at[rbase + r, pl.ds(POFF[b], SEG)],
                ssem.at[b, q]).wait()

        def prep(r, b, q):
            wait_refill(b, q)
            issue_gather(r, b, q)

        for b in range(2):
            for q in range(NQ):
                issue_refill(b, q)
        for b in range(2):
            prep(0, b, 0)
        for b in range(2):
            prep(1, b, 1)
        for b in range(2):
            wait_gather(0, b, 0)
            issue_store(0, b, 0)
            prep(2, b, 2)

        def step(r, q0, q2):
            for b in range(2):
                wait_gather(r - 1, b, q0)
                issue_store(r - 1, b, q0)
            for b in range(2):
                wait_store(r - 2, b, q2)
                issue_refill(b, q2)
            for b in range(2):
                prep(r + 1, b, q2)

        @pl.loop(2, RPW - 3, step=NQ)
        def _body(r0):
            for dr in range(NQ):
                step(r0 + dr, (dr + 1) % NQ, dr % NQ)

        step(RPW - 3, (RPW - 4) % NQ, (RPW - 2) % NQ)
        step(RPW - 2, (RPW - 3) % NQ, (RPW - 1) % NQ)

        for b in range(2):
            wait_gather(RPW - 2, b, (RPW - 2) % NQ)
            issue_store(RPW - 2, b, (RPW - 2) % NQ)
        for b in range(2):
            wait_gather(RPW - 1, b, (RPW - 1) % NQ)
            issue_store(RPW - 1, b, (RPW - 1) % NQ)
        for r in (RPW - 3, RPW - 2, RPW - 1):
            for b in range(2):
                wait_store(r, b, r % NQ)

    return emb


def kernel(x, token_table, pos_table):
    B, T = x.shape
    V, D = token_table.shape
    emb = _build(B, T, V, D)
    xi = x.astype(jnp.int32)
    segs = jnp.stack([xi[:, 0:SEG], xi[:, POFF[1]:T]], axis=1)
    x_seg = segs.reshape(NW, B // NW, 2, SEG)
    return emb(x_seg, token_table, pos_table)
